# baseline (speedup 1.0000x reference)
import functools

import jax
import jax.numpy as jnp
from jax import lax
from jax.experimental import pallas as pl
from jax.experimental.pallas import tpu as pltpu
from jax.experimental.pallas import tpu_sc as plsc

N = 10000
D = 128
E = 320000
FF = 2 * D

C = 128
EP = 327680
R = EP // C
NC = 2
NS = 16
NW = NC * NS
RPT = R // NW
NP = 10240
NPT = NP // NS
G = 4
NG = RPT // G


def _sc_segment_sum(x, ei_pad):
    z128 = jnp.zeros((NPT, D), jnp.bfloat16)
    z16 = jnp.zeros((NPT, 16), jnp.float32)
    ones_c = jnp.ones((C, 16), jnp.float32)

    mesh = plsc.VectorSubcoreMesh(core_axis_name="c", subcore_axis_name="s")

    @functools.partial(
        pl.kernel,
        mesh=mesh,
        out_type=(
            jax.ShapeDtypeStruct((NC, NP, D), jnp.bfloat16),
            jax.ShapeDtypeStruct((NC, NP, 16), jnp.float32),
        ),
        scratch_types=[
            pltpu.VMEM((2, G, C), jnp.int32),
            pltpu.VMEM((2, G, C), jnp.int32),
            pltpu.VMEM((2, G * C, D), jnp.bfloat16),
            pltpu.VMEM((C, 16), jnp.float32),
            pltpu.VMEM_SHARED((NP, D), jnp.bfloat16),
            pltpu.VMEM_SHARED((NP, 16), jnp.float32),
            pltpu.SemaphoreType.DMA,
            pltpu.SemaphoreType.DMA,
            pltpu.SemaphoreType.DMA,
            pltpu.SemaphoreType.DMA,
            pltpu.SemaphoreType.DMA,
        ],
        compiler_params=pltpu.CompilerParams(use_tc_tiling_on_sc=False),
    )
    def sc_kernel(x_hbm, ei_hbm, z128_hbm, z16_hbm, ones_hbm,
                  out_agg, out_deg, sidx, didx, buf_v, ones_v,
                  agg_sh, deg_sh, gsem, ssem, dsem, xssem, xdsem):
        cid = lax.axis_index("c")
        sid = lax.axis_index("s")
        wid = sid * NC + cid
        base = wid * RPT

        pltpu.sync_copy(ones_hbm, ones_v)
        nbase = sid * NPT
        pltpu.sync_copy(z128_hbm, agg_sh.at[pl.ds(nbase, NPT)])
        pltpu.sync_copy(z16_hbm, deg_sh.at[pl.ds(nbase, NPT)])

        dummy_rows = x_hbm.at[pl.ds(0, G * C)]
        dummy_ones = z16_hbm.at[pl.ds(0, C)]
        dummy_idx = ei_hbm.at[0, pl.ds(0, C)]

        def fire_idx(g, row, idx_buf, sem):
            for b in range(G):
                off = (base + g * G + b) * C
                pltpu.async_copy(ei_hbm.at[row, pl.ds(off, C)],
                                 idx_buf.at[b], sem)

        def sync_idx(g, row, idx_buf):
            for b in range(G):
                off = (base + g * G + b) * C
                pltpu.sync_copy(ei_hbm.at[row, pl.ds(off, C)], idx_buf.at[b])

        def fire_gathers(slot, dst_buf):
            for b in range(G):
                pltpu.async_copy(x_hbm.at[sidx.at[slot].at[b]],
                                 dst_buf.at[pl.ds(b * C, C)], gsem)

        def fire_scatters(slot, src_buf):
            for b in range(G):
                pltpu.async_copy(src_buf.at[pl.ds(b * C, C)],
                                 agg_sh.at[didx.at[slot].at[b]], ssem,
                                 add=True)
            for b in range(G):
                pltpu.async_copy(ones_v, deg_sh.at[didx.at[slot].at[b]],
                                 dsem, add=True)

        def drain_scatters():
            pltpu.make_async_copy(dummy_rows, buf_v.at[0], ssem).wait()
            for _b in range(G):
                pltpu.make_async_copy(dummy_ones, ones_v, dsem).wait()

        sync_idx(0, 0, sidx.at[0])
        sync_idx(0, 1, didx.at[0])
        if NG > 1:
            fire_idx(1, 0, sidx.at[1], xssem)
            fire_idx(1, 1, didx.at[1], xdsem)
        plsc.subcore_barrier()
        fire_gathers(0, buf_v.at[0])

        def group_body(g, carry):
            p = lax.rem(g, 2)
            cur = buf_v.at[p]
            nxt = buf_v.at[1 - p]
            pltpu.make_async_copy(dummy_rows, cur, gsem).wait()

            @pl.when(g >= 1)
            def _():
                drain_scatters()

                @pl.when(g + 1 < NG)
                def _():
                    fire_idx(g + 1, 1, didx.at[1 - p], xdsem)

            @pl.when(g + 1 < NG)
            def _():
                for _b in range(G):
                    pltpu.make_async_copy(dummy_idx, sidx.at[0].at[0],
                                          xssem).wait()

                @pl.when(g + 2 < NG)
                def _():
                    fire_idx(g + 2, 0, sidx.at[p], xssem)

                fire_gathers(1 - p, nxt)

            @pl.when(g >= 1)
            def _():
                for _b in range(G):
                    pltpu.make_async_copy(dummy_idx, didx.at[0].at[0],
                                          xdsem).wait()

            fire_scatters(p, cur)
            return carry

        lax.fori_loop(0, NG, group_body, 0)
        drain_scatters()
        plsc.subcore_barrier()

        pltpu.sync_copy(agg_sh.at[pl.ds(nbase, NPT)],
                        out_agg.at[cid, pl.ds(nbase, NPT)])
        pltpu.sync_copy(deg_sh.at[pl.ds(nbase, NPT)],
                        out_deg.at[cid, pl.ds(nbase, NPT)])

    return sc_kernel(x, ei_pad, z128, z16, ones_c)


def _ln(h, g, b, eps=1e-5):
    mu = jnp.mean(h, axis=-1, keepdims=True)
    var = jnp.mean((h - mu) ** 2, axis=-1, keepdims=True)
    return (h - mu) * lax.rsqrt(var + eps) * g + b


BN = 1000


def _tc_body(pagg, pdeg, x, Wm, bm, g1, b1, g2, b2, W1, bf1, W2, bf2, g3, b3,
             out):
    agg = pagg[0].astype(jnp.float32) + pagg[1].astype(jnp.float32)
    deg = pdeg[0, :, 0:1] + pdeg[1, :, 0:1]
    agg = agg / jnp.maximum(deg, 1.0)
    h = jnp.dot(agg, Wm[...], preferred_element_type=jnp.float32) + bm[...]
    h = _ln(h, g1[...], b1[...])
    h = jnp.maximum(h, 0.0) + x[...]
    res = h
    h2 = _ln(h, g2[...], b2[...])
    h2 = jnp.maximum(
        jnp.dot(h2, W1[...], preferred_element_type=jnp.float32) + bf1[...],
        0.0)
    h2 = jnp.dot(h2, W2[...], preferred_element_type=jnp.float32) + bf2[...]
    out[...] = _ln(h2 + res, g3[...], b3[...])


def _tc_dense(pagg, pdeg, x, Wm, bm, g1, b1, g2, b2, W1, bf1, W2, bf2, g3, b3):
    full = lambda shape: pl.BlockSpec(shape, lambda i: (0,) * len(shape))
    return pl.pallas_call(
        _tc_body,
        out_shape=jax.ShapeDtypeStruct((N, D), jnp.float32),
        grid=(N // BN,),
        in_specs=[
            pl.BlockSpec((NC, BN, D), lambda i: (0, i, 0)),
            pl.BlockSpec((NC, BN, 16), lambda i: (0, i, 0)),
            pl.BlockSpec((BN, D), lambda i: (i, 0)),
            full((D, D)), full((1, D)),
            full((1, D)), full((1, D)), full((1, D)), full((1, D)),
            full((D, FF)), full((1, FF)),
            full((FF, D)), full((1, D)),
            full((1, D)), full((1, D)),
        ],
        out_specs=pl.BlockSpec((BN, D), lambda i: (i, 0)),
    )(pagg, pdeg, x, Wm, bm, g1, b1, g2, b2, W1, bf1, W2, bf2, g3, b3)


def kernel(x, edge_index, W_mpnn, b_mpnn, ln1_g, ln1_b, ln2_g, ln2_b,
           W_ffn1, b_ffn1, W_ffn2, b_ffn2, ln3_g, ln3_b):
    pad_blk = jnp.concatenate(
        [jnp.zeros((1, EP - E), jnp.int32),
         jnp.full((1, EP - E), N, jnp.int32)])
    ei_pad = jnp.concatenate([edge_index, pad_blk], axis=1)
    pagg, pdeg = _sc_segment_sum(x.astype(jnp.bfloat16), ei_pad)
    r = lambda v: v.reshape(1, -1)
    return _tc_dense(pagg, pdeg, x, W_mpnn, r(b_mpnn), r(ln1_g), r(ln1_b),
                     r(ln2_g), r(ln2_b), W_ffn1, r(b_ffn1), W_ffn2, r(b_ffn2),
                     r(ln3_g), r(ln3_b))


# speedup vs baseline: 2.2064x; 2.2064x over previous
import functools

import jax
import jax.numpy as jnp
from jax import lax
from jax.experimental import pallas as pl
from jax.experimental.pallas import tpu as pltpu
from jax.experimental.pallas import tpu_sc as plsc

N = 10000
D = 128
E = 320000
FF = 2 * D

C = 125
R = E // C
NC = 2
NS = 16
NW = NC * NS
RPT = R // NW
NP = 10240
NPT = NP // NS
G = 4
NG = RPT // G


def _sc_segment_sum(x, src2d, dst2d):
    z128 = jnp.zeros((NPT, D), jnp.bfloat16)
    z16 = jnp.zeros((NPT, 16), jnp.float32)
    ones_c = jnp.ones((C, 16), jnp.float32)

    mesh = plsc.VectorSubcoreMesh(core_axis_name="c", subcore_axis_name="s")

    @functools.partial(
        pl.kernel,
        mesh=mesh,
        out_type=(
            jax.ShapeDtypeStruct((NC, NP, D), jnp.bfloat16),
            jax.ShapeDtypeStruct((NC, NP, 16), jnp.float32),
        ),
        scratch_types=[
            pltpu.VMEM((2, G, C), jnp.int32),
            pltpu.VMEM((2, G, C), jnp.int32),
            pltpu.VMEM((2, G * C, D), jnp.bfloat16),
            pltpu.VMEM((C, 16), jnp.float32),
            pltpu.VMEM_SHARED((NP, D), jnp.bfloat16),
            pltpu.VMEM_SHARED((NP, 16), jnp.float32),
            pltpu.SemaphoreType.DMA,
            pltpu.SemaphoreType.DMA,
            pltpu.SemaphoreType.DMA,
            pltpu.SemaphoreType.DMA,
            pltpu.SemaphoreType.DMA,
        ],
        compiler_params=pltpu.CompilerParams(use_tc_tiling_on_sc=False),
    )
    def sc_kernel(x_hbm, src_hbm, dst_hbm, z128_hbm, z16_hbm, ones_hbm,
                  out_agg, out_deg, sidx, didx, buf_v, ones_v,
                  agg_sh, deg_sh, gsem, ssem, dsem, xssem, xdsem):
        cid = lax.axis_index("c")
        sid = lax.axis_index("s")
        wid = sid * NC + cid
        base = wid * RPT

        pltpu.sync_copy(ones_hbm, ones_v)
        nbase = sid * NPT
        pltpu.sync_copy(z128_hbm, agg_sh.at[pl.ds(nbase, NPT)])
        pltpu.sync_copy(z16_hbm, deg_sh.at[pl.ds(nbase, NPT)])

        dummy_rows = x_hbm.at[pl.ds(0, G * C)]
        dummy_ones = z16_hbm.at[pl.ds(0, C)]
        dummy_idx = src_hbm.at[pl.ds(0, G)]

        def src_blk(g):
            return src_hbm.at[pl.ds(base + g * G, G)]

        def dst_blk(g):
            return dst_hbm.at[pl.ds(base + g * G, G)]

        def fire_gathers(slot, dst_buf):
            for b in range(G):
                pltpu.async_copy(x_hbm.at[sidx.at[slot].at[b]],
                                 dst_buf.at[pl.ds(b * C, C)], gsem)

        def fire_scatters(slot, src_buf):
            for b in range(G):
                pltpu.async_copy(src_buf.at[pl.ds(b * C, C)],
                                 agg_sh.at[didx.at[slot].at[b]], ssem,
                                 add=True)
            for b in range(G):
                pltpu.async_copy(ones_v, deg_sh.at[didx.at[slot].at[b]],
                                 dsem, add=True)

        def drain_scatters():
            pltpu.make_async_copy(dummy_rows, buf_v.at[0], ssem).wait()
            for _b in range(G):
                pltpu.make_async_copy(dummy_ones, ones_v, dsem).wait()

        pltpu.sync_copy(src_blk(0), sidx.at[0])
        pltpu.sync_copy(dst_blk(0), didx.at[0])
        if NG > 1:
            pltpu.async_copy(src_blk(1), sidx.at[1], xssem)
            pltpu.async_copy(dst_blk(1), didx.at[1], xdsem)
        plsc.subcore_barrier()
        fire_gathers(0, buf_v.at[0])

        def group_body(g, carry):
            p = lax.rem(g, 2)
            cur = buf_v.at[p]
            nxt = buf_v.at[1 - p]
            pltpu.make_async_copy(dummy_rows, cur, gsem).wait()

            @pl.when(g >= 1)
            def _():
                drain_scatters()

                @pl.when(g + 1 < NG)
                def _():
                    pltpu.async_copy(dst_blk(g + 1), didx.at[1 - p], xdsem)

            @pl.when(g + 1 < NG)
            def _():
                pltpu.make_async_copy(dummy_idx, sidx.at[0], xssem).wait()

                @pl.when(g + 2 < NG)
                def _():
                    pltpu.async_copy(src_blk(g + 2), sidx.at[p], xssem)

                fire_gathers(1 - p, nxt)

            @pl.when(g >= 1)
            def _():
                pltpu.make_async_copy(dummy_idx, didx.at[0], xdsem).wait()

            fire_scatters(p, cur)
            return carry

        lax.fori_loop(0, NG, group_body, 0)
        drain_scatters()
        plsc.subcore_barrier()

        pltpu.sync_copy(agg_sh.at[pl.ds(nbase, NPT)],
                        out_agg.at[cid, pl.ds(nbase, NPT)])
        pltpu.sync_copy(deg_sh.at[pl.ds(nbase, NPT)],
                        out_deg.at[cid, pl.ds(nbase, NPT)])

    return sc_kernel(x, src2d, dst2d, z128, z16, ones_c)


def _ln(h, g, b, eps=1e-5):
    mu = jnp.mean(h, axis=-1, keepdims=True)
    var = jnp.mean((h - mu) ** 2, axis=-1, keepdims=True)
    return (h - mu) * lax.rsqrt(var + eps) * g + b


BN = 1000


def _tc_body(pagg, pdeg, x, Wm, bm, g1, b1, g2, b2, W1, bf1, W2, bf2, g3, b3,
             out):
    agg = pagg[0].astype(jnp.float32) + pagg[1].astype(jnp.float32)
    deg = pdeg[0, :, 0:1] + pdeg[1, :, 0:1]
    agg = agg / jnp.maximum(deg, 1.0)
    h = jnp.dot(agg, Wm[...], preferred_element_type=jnp.float32) + bm[...]
    h = _ln(h, g1[...], b1[...])
    h = jnp.maximum(h, 0.0) + x[...]
    res = h
    h2 = _ln(h, g2[...], b2[...])
    h2 = jnp.maximum(
        jnp.dot(h2, W1[...], preferred_element_type=jnp.float32) + bf1[...],
        0.0)
    h2 = jnp.dot(h2, W2[...], preferred_element_type=jnp.float32) + bf2[...]
    out[...] = _ln(h2 + res, g3[...], b3[...])


def _tc_dense(pagg, pdeg, x, Wm, bm, g1, b1, g2, b2, W1, bf1, W2, bf2, g3, b3):
    full = lambda shape: pl.BlockSpec(shape, lambda i: (0,) * len(shape))
    return pl.pallas_call(
        _tc_body,
        out_shape=jax.ShapeDtypeStruct((N, D), jnp.float32),
        grid=(N // BN,),
        in_specs=[
            pl.BlockSpec((NC, BN, D), lambda i: (0, i, 0)),
            pl.BlockSpec((NC, BN, 16), lambda i: (0, i, 0)),
            pl.BlockSpec((BN, D), lambda i: (i, 0)),
            full((D, D)), full((1, D)),
            full((1, D)), full((1, D)), full((1, D)), full((1, D)),
            full((D, FF)), full((1, FF)),
            full((FF, D)), full((1, D)),
            full((1, D)), full((1, D)),
        ],
        out_specs=pl.BlockSpec((BN, D), lambda i: (i, 0)),
    )(pagg, pdeg, x, Wm, bm, g1, b1, g2, b2, W1, bf1, W2, bf2, g3, b3)


def kernel(x, edge_index, W_mpnn, b_mpnn, ln1_g, ln1_b, ln2_g, ln2_b,
           W_ffn1, b_ffn1, W_ffn2, b_ffn2, ln3_g, ln3_b):
    src2d = edge_index[0].reshape(R, C)
    dst2d = edge_index[1].reshape(R, C)
    pagg, pdeg = _sc_segment_sum(x.astype(jnp.bfloat16), src2d, dst2d)
    r = lambda v: v.reshape(1, -1)
    return _tc_dense(pagg, pdeg, x, W_mpnn, r(b_mpnn), r(ln1_g), r(ln1_b),
                     r(ln2_g), r(ln2_b), W_ffn1, r(b_ffn1), W_ffn2, r(b_ffn2),
                     r(ln3_g), r(ln3_b))


# speedup vs baseline: 2.4336x; 1.1030x over previous
import functools

import jax
import jax.numpy as jnp
from jax import lax
from jax.experimental import pallas as pl
from jax.experimental.pallas import tpu as pltpu
from jax.experimental.pallas import tpu_sc as plsc

N = 10000
D = 128
E = 320000
FF = 2 * D

C = 125
R = E // C
NC = 2
NS = 16
NW = NC * NS
RPT = R // NW
NP = 10240
NPT = NP // NS
G = 4
NG = RPT // G


def _sc_segment_sum(x, ei3):
    z128 = jnp.zeros((NPT, D), jnp.bfloat16)
    z16 = jnp.zeros((NPT, 16), jnp.float32)
    ones_c = jnp.ones((C, 16), jnp.float32)

    mesh = plsc.VectorSubcoreMesh(core_axis_name="c", subcore_axis_name="s")

    @functools.partial(
        pl.kernel,
        mesh=mesh,
        out_type=(
            jax.ShapeDtypeStruct((NC, NP, D), jnp.bfloat16),
            jax.ShapeDtypeStruct((NC, NP, 16), jnp.float32),
        ),
        scratch_types=[
            pltpu.VMEM((2, G, C), jnp.int32),
            pltpu.VMEM((2, G, C), jnp.int32),
            pltpu.VMEM((2, G * C, D), jnp.bfloat16),
            pltpu.VMEM((C, 16), jnp.float32),
            pltpu.VMEM_SHARED((NP, D), jnp.bfloat16),
            pltpu.VMEM_SHARED((NP, 16), jnp.float32),
            pltpu.SemaphoreType.DMA,
            pltpu.SemaphoreType.DMA,
            pltpu.SemaphoreType.DMA,
            pltpu.SemaphoreType.DMA,
            pltpu.SemaphoreType.DMA,
        ],
        compiler_params=pltpu.CompilerParams(use_tc_tiling_on_sc=False),
    )
    def sc_kernel(x_hbm, ei_hbm, z128_hbm, z16_hbm, ones_hbm,
                  out_agg, out_deg, sidx, didx, buf_v, ones_v,
                  agg_sh, deg_sh, gsem, ssem, dsem, xssem, xdsem):
        src_hbm = ei_hbm.at[0]
        dst_hbm = ei_hbm.at[1]
        cid = lax.axis_index("c")
        sid = lax.axis_index("s")
        wid = sid * NC + cid
        base = wid * RPT

        pltpu.sync_copy(ones_hbm, ones_v)
        nbase = sid * NPT
        pltpu.sync_copy(z128_hbm, agg_sh.at[pl.ds(nbase, NPT)])
        pltpu.sync_copy(z16_hbm, deg_sh.at[pl.ds(nbase, NPT)])

        dummy_rows = x_hbm.at[pl.ds(0, G * C)]
        dummy_ones = z16_hbm.at[pl.ds(0, C)]
        dummy_idx = src_hbm.at[pl.ds(0, G)]

        def src_blk(g):
            return src_hbm.at[pl.ds(base + g * G, G)]

        def dst_blk(g):
            return dst_hbm.at[pl.ds(base + g * G, G)]

        def fire_gathers(slot, dst_buf):
            for b in range(G):
                pltpu.async_copy(x_hbm.at[sidx.at[slot].at[b]],
                                 dst_buf.at[pl.ds(b * C, C)], gsem)

        def fire_scatters(slot, src_buf):
            for b in range(G):
                pltpu.async_copy(src_buf.at[pl.ds(b * C, C)],
                                 agg_sh.at[didx.at[slot].at[b]], ssem,
                                 add=True)
            for b in range(G):
                pltpu.async_copy(ones_v, deg_sh.at[didx.at[slot].at[b]],
                                 dsem, add=True)

        def drain_scatters():
            pltpu.make_async_copy(dummy_rows, buf_v.at[0], ssem).wait()
            for _b in range(G):
                pltpu.make_async_copy(dummy_ones, ones_v, dsem).wait()

        pltpu.sync_copy(src_blk(0), sidx.at[0])
        pltpu.sync_copy(dst_blk(0), didx.at[0])
        if NG > 1:
            pltpu.async_copy(src_blk(1), sidx.at[1], xssem)
            pltpu.async_copy(dst_blk(1), didx.at[1], xdsem)
        plsc.subcore_barrier()
        fire_gathers(0, buf_v.at[0])

        def group_body(g, carry):
            p = lax.rem(g, 2)
            cur = buf_v.at[p]
            nxt = buf_v.at[1 - p]
            pltpu.make_async_copy(dummy_rows, cur, gsem).wait()

            @pl.when(g >= 1)
            def _():
                drain_scatters()

                @pl.when(g + 1 < NG)
                def _():
                    pltpu.async_copy(dst_blk(g + 1), didx.at[1 - p], xdsem)

            @pl.when(g + 1 < NG)
            def _():
                pltpu.make_async_copy(dummy_idx, sidx.at[0], xssem).wait()

                @pl.when(g + 2 < NG)
                def _():
                    pltpu.async_copy(src_blk(g + 2), sidx.at[p], xssem)

                fire_gathers(1 - p, nxt)

            @pl.when(g >= 1)
            def _():
                pltpu.make_async_copy(dummy_idx, didx.at[0], xdsem).wait()

            fire_scatters(p, cur)
            return carry

        lax.fori_loop(0, NG, group_body, 0)
        drain_scatters()
        plsc.subcore_barrier()

        pltpu.sync_copy(agg_sh.at[pl.ds(nbase, NPT)],
                        out_agg.at[cid, pl.ds(nbase, NPT)])
        pltpu.sync_copy(deg_sh.at[pl.ds(nbase, NPT)],
                        out_deg.at[cid, pl.ds(nbase, NPT)])

    return sc_kernel(x, ei3, z128, z16, ones_c)


def _ln(h, g, b, eps=1e-5):
    mu = jnp.mean(h, axis=-1, keepdims=True)
    var = jnp.mean((h - mu) ** 2, axis=-1, keepdims=True)
    return (h - mu) * lax.rsqrt(var + eps) * g + b


BN = 2000


def _tc_body(pagg, pdeg, x, Wm, bm, g1, b1, g2, b2, W1, bf1, W2, bf2, g3, b3,
             out):
    agg = pagg[0].astype(jnp.float32) + pagg[1].astype(jnp.float32)
    deg = pdeg[0, :, 0:1] + pdeg[1, :, 0:1]
    agg = agg / jnp.maximum(deg, 1.0)
    h = jnp.dot(agg, Wm[...], preferred_element_type=jnp.float32) + bm[...]
    h = _ln(h, g1[...], b1[...])
    h = jnp.maximum(h, 0.0) + x[...]
    res = h
    h2 = _ln(h, g2[...], b2[...])
    h2 = jnp.maximum(
        jnp.dot(h2, W1[...], preferred_element_type=jnp.float32) + bf1[...],
        0.0)
    h2 = jnp.dot(h2, W2[...], preferred_element_type=jnp.float32) + bf2[...]
    out[...] = _ln(h2 + res, g3[...], b3[...])


def _tc_dense(pagg, pdeg, x, Wm, bm, g1, b1, g2, b2, W1, bf1, W2, bf2, g3, b3):
    full = lambda shape: pl.BlockSpec(shape, lambda i: (0,) * len(shape))
    return pl.pallas_call(
        _tc_body,
        out_shape=jax.ShapeDtypeStruct((N, D), jnp.float32),
        grid=(N // BN,),
        in_specs=[
            pl.BlockSpec((NC, BN, D), lambda i: (0, i, 0)),
            pl.BlockSpec((NC, BN, 16), lambda i: (0, i, 0)),
            pl.BlockSpec((BN, D), lambda i: (i, 0)),
            full((D, D)), full((1, D)),
            full((1, D)), full((1, D)), full((1, D)), full((1, D)),
            full((D, FF)), full((1, FF)),
            full((FF, D)), full((1, D)),
            full((1, D)), full((1, D)),
        ],
        out_specs=pl.BlockSpec((BN, D), lambda i: (i, 0)),
    )(pagg, pdeg, x, Wm, bm, g1, b1, g2, b2, W1, bf1, W2, bf2, g3, b3)


def kernel(x, edge_index, W_mpnn, b_mpnn, ln1_g, ln1_b, ln2_g, ln2_b,
           W_ffn1, b_ffn1, W_ffn2, b_ffn2, ln3_g, ln3_b):
    ei3 = edge_index.reshape(2, R, C)
    pagg, pdeg = _sc_segment_sum(x.astype(jnp.bfloat16), ei3)
    r = lambda v: v.reshape(1, -1)
    return _tc_dense(pagg, pdeg, x, W_mpnn, r(b_mpnn), r(ln1_g), r(ln1_b),
                     r(ln2_g), r(ln2_b), W_ffn1, r(b_ffn1), W_ffn2, r(b_ffn2),
                     r(ln3_g), r(ln3_b))
